# baseline (speedup 1.0000x reference)
import functools

import jax
import jax.numpy as jnp
from jax import lax
from jax.experimental import pallas as pl
from jax.experimental.pallas import tpu as pltpu
from jax.experimental.pallas import tpu_sc as plsc

_MAX_ATTENTION_SIZE = 4096
_SINK = 64


def kernel(cache_k, cache_v, k, v, global_end_index, local_end_index, num_new_tokens):
    BS, S, H, D = cache_k.shape
    NN = k.shape[1]
    F = H * D
    SF = S * F
    NR = S - NN - _SINK
    HALF = NR // 2

    lei = jnp.asarray(local_end_index, jnp.int32)
    nnt = jnp.asarray(num_new_tokens, jnp.int32)
    num_evicted = lei + nnt - S
    src0 = jnp.clip(_SINK + num_evicted, 0, S - NR).astype(jnp.int32)
    new_local_end = (lei + nnt - num_evicted).astype(jnp.int32)
    window_start = jnp.maximum(new_local_end - _MAX_ATTENTION_SIZE, 0).astype(jnp.int32)

    ck = cache_k.reshape(BS * SF)
    cv = cache_v.reshape(BS * SF)
    kn = k.reshape(BS * NN * F)
    vn = v.reshape(BS * NN * F)
    src0_v = jnp.full((16,), src0, jnp.int32)

    mesh = plsc.VectorSubcoreMesh(core_axis_name="c", subcore_axis_name="s")

    @functools.partial(
        pl.kernel,
        out_type=(
            jax.ShapeDtypeStruct((BS * SF,), jnp.float32),
            jax.ShapeDtypeStruct((BS * SF,), jnp.float32),
        ),
        mesh=mesh,
        scratch_types=[pltpu.VMEM((16,), jnp.int32)],
    )
    def _copy(ck_h, cv_h, kn_h, vn_h, s0_h, ok_h, ov_h, s0_vm):
        c = lax.axis_index("c")
        s = lax.axis_index("s")
        b = s // 2
        half = s % 2
        pltpu.sync_copy(s0_h, s0_vm)
        s0 = s0_vm[...][0]
        src_off = pl.multiple_of(b * SF + (s0 + half * HALF) * F, 8)
        dst_off = pl.multiple_of(b * SF + (_SINK + half * HALF) * F, 8)
        sink_off = pl.multiple_of(b * SF, 8)
        new_src = pl.multiple_of(b * (NN * F), 8)
        new_dst = pl.multiple_of(b * SF + (S - NN) * F, 8)

        def do(src_h, new_h, out_h):
            pltpu.sync_copy(src_h.at[pl.ds(src_off, HALF * F)],
                            out_h.at[pl.ds(dst_off, HALF * F)])

            @pl.when(half == 0)
            def _():
                pltpu.sync_copy(src_h.at[pl.ds(sink_off, _SINK * F)],
                                out_h.at[pl.ds(sink_off, _SINK * F)])

            @pl.when(half == 1)
            def _():
                pltpu.sync_copy(new_h.at[pl.ds(new_src, NN * F)],
                                out_h.at[pl.ds(new_dst, NN * F)])

        @pl.when(c == 0)
        def _():
            do(ck_h, kn_h, ok_h)

        @pl.when(c == 1)
        def _():
            do(cv_h, vn_h, ov_h)

    ok, ov = _copy(ck, cv, kn, vn, src0_v)
    return (ok.reshape(BS, S, H, D), ov.reshape(BS, S, H, D),
            window_start, new_local_end)


# speedup vs baseline: 6.5658x; 6.5658x over previous
import functools

import jax
import jax.numpy as jnp
from jax import lax
from jax.experimental import pallas as pl
from jax.experimental.pallas import tpu as pltpu
from jax.experimental.pallas import tpu_sc as plsc

_MAX_ATTENTION_SIZE = 4096
_SINK = 64


def kernel(cache_k, cache_v, k, v, global_end_index, local_end_index, num_new_tokens):
    BS, S, H, D = cache_k.shape
    NN = k.shape[1]
    NR = S - NN - _SINK
    HALF = NR // 2
    C = 25
    NCH = HALF // C
    assert HALF % C == 0

    lei = jnp.asarray(local_end_index, jnp.int32)
    nnt = jnp.asarray(num_new_tokens, jnp.int32)
    num_evicted = lei + nnt - S
    src0 = jnp.clip(_SINK + num_evicted, 0, S - NR).astype(jnp.int32)
    new_local_end = (lei + nnt - num_evicted).astype(jnp.int32)
    window_start = jnp.maximum(new_local_end - _MAX_ATTENTION_SIZE, 0).astype(jnp.int32)

    src0_v = jnp.full((16,), src0, jnp.int32)

    mesh = plsc.VectorSubcoreMesh(core_axis_name="c", subcore_axis_name="s")

    @functools.partial(
        pl.kernel,
        out_type=(
            jax.ShapeDtypeStruct((BS, S, H, D), jnp.float32),
            jax.ShapeDtypeStruct((BS, S, H, D), jnp.float32),
        ),
        mesh=mesh,
        scratch_types=[
            pltpu.VMEM((C, H, D), jnp.float32),
            pltpu.VMEM((C, H, D), jnp.float32),
            pltpu.VMEM((16,), jnp.int32),
            pltpu.SemaphoreType.DMA,
            pltpu.SemaphoreType.DMA,
            pltpu.SemaphoreType.DMA,
            pltpu.SemaphoreType.DMA,
        ],
    )
    def _copy(ck_h, cv_h, kn_h, vn_h, s0_h, ok_h, ov_h,
              buf0, buf1, s0_vm, gs0, gs1, ss0, ss1):
        c = lax.axis_index("c")
        s = lax.axis_index("s")
        b = s // 2
        half = s % 2
        pltpu.sync_copy(s0_h, s0_vm)
        s0 = s0_vm[...][0]
        src_base = s0 + half * HALF
        dst_base = _SINK + half * HALF
        bufs = (buf0, buf1)
        gsems = (gs0, gs1)
        ssems = (ss0, ss1)

        def do(src_h, new_h, out_h):
            def gather(i, buf, sem):
                return pltpu.make_async_copy(
                    src_h.at[b, pl.ds(src_base + i * C, C)], buf, sem)

            def scatter(i, buf, sem):
                return pltpu.make_async_copy(
                    buf, out_h.at[b, pl.ds(dst_base + i * C, C)], sem)

            gather(0, bufs[0], gsems[0]).start()
            for i in range(NCH):
                cur = i & 1
                nxt = 1 - cur
                if i + 1 < NCH:
                    if i >= 1:
                        scatter(i - 1, bufs[nxt], ssems[nxt]).wait()
                    gather(i + 1, bufs[nxt], gsems[nxt]).start()
                gather(i, bufs[cur], gsems[cur]).wait()
                scatter(i, bufs[cur], ssems[cur]).start()
            if NCH >= 2:
                scatter(NCH - 2, bufs[(NCH - 2) & 1], ssems[(NCH - 2) & 1]).wait()
            scatter(NCH - 1, bufs[(NCH - 1) & 1], ssems[(NCH - 1) & 1]).wait()

            def tail(tsrc_h, src_row, dst_row, nrows):
                off = 0
                while off < nrows:
                    m = min(C, nrows - off)
                    pltpu.sync_copy(tsrc_h.at[b, pl.ds(src_row + off, m)],
                                    buf0.at[pl.ds(0, m)])
                    pltpu.sync_copy(buf0.at[pl.ds(0, m)],
                                    out_h.at[b, pl.ds(dst_row + off, m)])
                    off += m

            @pl.when(half == 0)
            def _():
                tail(src_h, 0, 0, _SINK)

            @pl.when(half == 1)
            def _():
                tail(new_h, 0, S - NN, NN)

        @pl.when(c == 0)
        def _():
            do(ck_h, kn_h, ok_h)

        @pl.when(c == 1)
        def _():
            do(cv_h, vn_h, ov_h)

    ok, ov = _copy(cache_k, cache_v, k, v, src0_v)
    return (ok, ov, window_start, new_local_end)


# speedup vs baseline: 6.6186x; 1.0080x over previous
import functools

import jax
import jax.numpy as jnp
from jax import lax
from jax.experimental import pallas as pl
from jax.experimental.pallas import tpu as pltpu
from jax.experimental.pallas import tpu_sc as plsc

_MAX_ATTENTION_SIZE = 4096
_SINK = 64


def kernel(cache_k, cache_v, k, v, global_end_index, local_end_index, num_new_tokens):
    BS, S, H, D = cache_k.shape
    NN = k.shape[1]
    NR = S - NN - _SINK
    HALF = NR // 2
    C = 40
    NCH = HALF // C
    assert HALF % C == 0

    lei = jnp.asarray(local_end_index, jnp.int32)
    nnt = jnp.asarray(num_new_tokens, jnp.int32)
    num_evicted = lei + nnt - S
    src0 = jnp.clip(_SINK + num_evicted, 0, S - NR).astype(jnp.int32)
    new_local_end = (lei + nnt - num_evicted).astype(jnp.int32)
    window_start = jnp.maximum(new_local_end - _MAX_ATTENTION_SIZE, 0).astype(jnp.int32)

    src0_v = jnp.full((16,), src0, jnp.int32)

    mesh = plsc.VectorSubcoreMesh(core_axis_name="c", subcore_axis_name="s")

    @functools.partial(
        pl.kernel,
        out_type=(
            jax.ShapeDtypeStruct((BS, S, H, D), jnp.float32),
            jax.ShapeDtypeStruct((BS, S, H, D), jnp.float32),
        ),
        mesh=mesh,
        scratch_types=[
            pltpu.VMEM((16,), jnp.int32),
            pltpu.SemaphoreType.DMA,
            pltpu.SemaphoreType.DMA,
            pltpu.SemaphoreType.DMA,
            pltpu.SemaphoreType.DMA,
        ],
    )
    def _copy(ck_h, cv_h, kn_h, vn_h, s0_h, ok_h, ov_h,
              s0_vm, gs0, gs1, ss0, ss1):
        c = lax.axis_index("c")
        s = lax.axis_index("s")
        b = s // 2
        half = s % 2
        pltpu.sync_copy(s0_h, s0_vm)
        s0 = s0_vm[...][0]
        src_base = s0 + half * HALF
        dst_base = _SINK + half * HALF
        gsems = (gs0, gs1)
        ssems = (ss0, ss1)

        def do(src_h, new_h, out_h):
            def scoped(buf0, buf1):
                do_bufs(src_h, new_h, out_h, buf0, buf1)

            pl.run_scoped(scoped,
                          pltpu.VMEM((C, H, D), jnp.float32),
                          pltpu.VMEM((C, H, D), jnp.float32))

        def do_bufs(src_h, new_h, out_h, buf0, buf1):
            bufs = (buf0, buf1)

            def gather(i, buf, sem):
                return pltpu.make_async_copy(
                    src_h.at[b, pl.ds(src_base + i * C, C)], buf, sem)

            def scatter(i, buf, sem):
                return pltpu.make_async_copy(
                    buf, out_h.at[b, pl.ds(dst_base + i * C, C)], sem)

            gather(0, bufs[0], gsems[0]).start()
            for i in range(NCH):
                cur = i & 1
                nxt = 1 - cur
                if i + 1 < NCH:
                    if i >= 1:
                        scatter(i - 1, bufs[nxt], ssems[nxt]).wait()
                    gather(i + 1, bufs[nxt], gsems[nxt]).start()
                gather(i, bufs[cur], gsems[cur]).wait()
                scatter(i, bufs[cur], ssems[cur]).start()
            if NCH >= 2:
                scatter(NCH - 2, bufs[(NCH - 2) & 1], ssems[(NCH - 2) & 1]).wait()
            scatter(NCH - 1, bufs[(NCH - 1) & 1], ssems[(NCH - 1) & 1]).wait()

            def tail(tsrc_h, src_row, dst_row, nrows):
                off = 0
                while off < nrows:
                    m = min(C, nrows - off)
                    pltpu.sync_copy(tsrc_h.at[b, pl.ds(src_row + off, m)],
                                    buf0.at[pl.ds(0, m)])
                    pltpu.sync_copy(buf0.at[pl.ds(0, m)],
                                    out_h.at[b, pl.ds(dst_row + off, m)])
                    off += m

            @pl.when(half == 0)
            def _():
                tail(src_h, 0, 0, _SINK)

            @pl.when(half == 1)
            def _():
                tail(new_h, 0, S - NN, NN)

        @pl.when(c == 0)
        def _():
            do(ck_h, kn_h, ok_h)

        @pl.when(c == 1)
        def _():
            do(cv_h, vn_h, ov_h)

    ok, ov = _copy(cache_k, cache_v, k, v, src0_v)
    return (ok, ov, window_start, new_local_end)
